# Initial kernel scaffold; baseline (speedup 1.0000x reference)
import jax
import jax.numpy as jnp
from jax.experimental import pallas as pl


def kernel(event_values, event_indices, use_soft):
    raise NotImplementedError("write your pallas kernel here")


# speedup vs baseline: 1.3228x; 1.3228x over previous
import functools

import jax
import jax.numpy as jnp
from jax import lax
from jax.experimental import pallas as pl
from jax.experimental.pallas import tpu as pltpu
from jax.experimental.pallas import tpu_sc as plsc

_FRAME_NUMBER = 16
_FRAME_SIZE = 128
_SAMPLE_NUM = 8
_NUM_EVENTS = 1048576

_NBINS = _FRAME_SIZE * _FRAME_SIZE
_TW = _NUM_EVENTS // _FRAME_NUMBER
_CHUNK = 8192
_ACC = _NBINS * 4


def _sc_body(ev_hbm, idx1_hbm, idx2_hbm, out_hbm, bidx1, bidx2, bv0, bv1, acc):
    nc = 2
    wid = lax.axis_index("s") * nc + lax.axis_index("c")
    pair = wid % 4
    fbase = (wid // 4) * 2

    zeros16 = jnp.zeros((16,), jnp.float32)

    for fi in range(2):
        f = fbase + fi

        def _zero(i, _):
            acc[pl.ds(i * 16, 16)] = zeros16
            return ()

        lax.fori_loop(0, _ACC // 16, _zero, (), unroll=4)

        def _chunk(cidx, _):
            off = f * _TW + cidx * _CHUNK
            pltpu.sync_copy(idx1_hbm.at[pl.ds(off, _CHUNK)], bidx1)
            pltpu.sync_copy(idx2_hbm.at[pl.ds(off, _CHUNK)], bidx2)
            v_off = (2 * pair) * _NUM_EVENTS + off
            pltpu.sync_copy(ev_hbm.at[pl.ds(v_off, _CHUNK)], bv0)
            pltpu.sync_copy(ev_hbm.at[pl.ds(v_off + _NUM_EVENTS, _CHUNK)], bv1)

            def _group(i, _):
                s = pl.ds(i * 16, 16)
                b = (bidx2[s] * _FRAME_SIZE + bidx1[s]) * 4
                v0 = bv0[s]
                v1 = bv1[s]
                plsc.addupdate_scatter(acc, [b], jnp.maximum(-v0, 0.0))
                plsc.addupdate_scatter(acc, [b + 1], jnp.maximum(v0, 0.0))
                plsc.addupdate_scatter(acc, [b + 2], jnp.maximum(-v1, 0.0))
                plsc.addupdate_scatter(acc, [b + 3], jnp.maximum(v1, 0.0))
                return ()

            lax.fori_loop(0, _CHUNK // 16, _group, (), unroll=2)
            return ()

        lax.fori_loop(0, _TW // _CHUNK, _chunk, ())
        pltpu.sync_copy(acc, out_hbm.at[pl.ds((f * 4 + pair) * _ACC, _ACC)])


@jax.jit
def _frame_hist(ev_flat, idx1, idx2):
    mesh = plsc.VectorSubcoreMesh(core_axis_name="c", subcore_axis_name="s")
    return pl.kernel(
        _sc_body,
        out_type=jax.ShapeDtypeStruct((_FRAME_NUMBER * 4 * _ACC,), jnp.float32),
        mesh=mesh,
        compiler_params=pltpu.CompilerParams(needs_layout_passes=False),
        scratch_types=[
            pltpu.VMEM((_CHUNK,), jnp.int32),
            pltpu.VMEM((_CHUNK,), jnp.int32),
            pltpu.VMEM((_CHUNK,), jnp.float32),
            pltpu.VMEM((_CHUNK,), jnp.float32),
            pltpu.VMEM((_ACC,), jnp.float32),
        ],
    )(ev_flat, idx1, idx2)


def kernel(event_values, event_indices, use_soft):
    ev = jnp.where(use_soft, 0.0, event_values).reshape(-1)
    idx1 = event_indices[0, 1]
    idx2 = event_indices[0, 2]
    raw = _frame_hist(ev, idx1, idx2)
    out = raw.reshape(_FRAME_NUMBER, 4, _FRAME_SIZE, _FRAME_SIZE, 2, 2)
    out = out.transpose(0, 1, 4, 5, 2, 3)
    return out.reshape(_FRAME_NUMBER, _SAMPLE_NUM, 2, _FRAME_SIZE, _FRAME_SIZE)
